# baseline (speedup 1.0000x reference)
import functools

import jax
import jax.numpy as jnp
from jax import lax
from jax.experimental import pallas as pl
from jax.experimental.pallas import tpu as pltpu
from jax.experimental.pallas import tpu_sc as plsc

N_TOK = 32768
D = 4
B = 16
C = 128
NW = 32
K = N_TOK // (C * NW)

_mesh = plsc.VectorSubcoreMesh(core_axis_name="c", subcore_axis_name="s")


@functools.partial(
    pl.kernel,
    mesh=_mesh,
    out_type=jax.ShapeDtypeStruct((N_TOK, D), jnp.float32),
    compiler_params=pltpu.CompilerParams(use_tc_tiling_on_sc=False),
    scratch_types=[
        pltpu.VMEM((17,), jnp.int32),
        pltpu.VMEM((C, 16), jnp.float32),
        pltpu.VMEM((2 * C, 16), jnp.float32),
        pltpu.VMEM((2 * C, 16), jnp.float32),
    ],
)
def _normalize_sc(x_hbm, ptr_hbm, mean_hbm, std_hbm, out_hbm,
                  ptr_v, x16, m16, s16):
    wid = lax.axis_index("s") * 2 + lax.axis_index("c")
    pltpu.sync_copy(ptr_hbm, ptr_v)
    pv = ptr_v[pl.ds(0, 16)]
    starts = [jnp.int32(0)] + [pv[s] for s in range(1, B)]
    ends = starts[1:] + [jnp.int32(N_TOK)]

    def chunk_body(k, carry):
        c0 = (wid * K + k) * C
        pltpu.sync_copy(x_hbm.at[pl.ds(c0, C)], x16.at[:, pl.ds(0, 4)])

        for s in range(B):
            start_s = starts[s]

            @pl.when(jnp.logical_and(start_s < c0 + C, ends[s] > c0))
            def _():
                d0 = jnp.maximum(start_s - c0, 0)
                src0 = jnp.maximum(c0 - start_s, 0)
                pltpu.sync_copy(mean_hbm.at[pl.ds(src0, C)],
                                m16.at[pl.ds(d0, C), pl.ds(0, 4)])
                pltpu.sync_copy(std_hbm.at[pl.ds(src0, C)],
                                s16.at[pl.ds(d0, C), pl.ds(0, 4)])

        def body(p, carry2):
            x16[p] = (x16[p] - m16[p]) / s16[p]
            return carry2

        lax.fori_loop(0, C, body, 0)
        pltpu.sync_copy(x16.at[:, pl.ds(0, 4)], out_hbm.at[pl.ds(c0, C)])
        return carry

    lax.fori_loop(0, K, chunk_body, 0)


def kernel(x, ptr, mean, std):
    return _normalize_sc(x, ptr.astype(jnp.int32), mean, std)
